# Initial kernel scaffold; baseline (speedup 1.0000x reference)
import jax
import jax.numpy as jnp
from jax.experimental import pallas as pl


def kernel(node_0, node_1, coord, idx_i, idx_j, W0, b0, W1, Wr0, br0, Wr1, br1):
    raise NotImplementedError("write your pallas kernel here")


# speedup vs baseline: 1.3876x; 1.3876x over previous
import functools
import jax
import jax.numpy as jnp
from jax import lax
from jax.experimental import pallas as pl
from jax.experimental.pallas import tpu as pltpu
from jax.experimental.pallas import tpu_sc as plsc

CUTOFF = 5.0
GAMMA = 10.0


def _table_body(n0_ref, n1_ref, cp_ref, w0_ref, b0_ref, w1e_ref, t_ref):
    x0 = jnp.dot(n0_ref[...], w0_ref[...], preferred_element_type=jnp.float32)
    x0 = x0 + b0_ref[...]
    x1 = jnp.dot(n1_ref[...], w1e_ref[...], preferred_element_type=jnp.float32)
    t_ref[...] = jnp.concatenate([x0, x1, cp_ref[...]], axis=1)


def _build_table(n0, n1f, coordp, w0, b0, w1e, bn):
    n = n0.shape[0]
    c_out = w0.shape[1]
    width = c_out * 4 + 16
    grid = n // bn
    return pl.pallas_call(
        _table_body,
        grid=(grid,),
        in_specs=[
            pl.BlockSpec((bn, n0.shape[1]), lambda i: (i, 0)),
            pl.BlockSpec((bn, n1f.shape[1]), lambda i: (i, 0)),
            pl.BlockSpec((bn, 16), lambda i: (i, 0)),
            pl.BlockSpec(w0.shape, lambda i: (0, 0)),
            pl.BlockSpec((1, c_out), lambda i: (0, 0)),
            pl.BlockSpec(w1e.shape, lambda i: (0, 0)),
        ],
        out_specs=pl.BlockSpec((bn, width), lambda i: (i, 0)),
        out_shape=jax.ShapeDtypeStruct((n, width), jnp.float32),
    )(n0, n1f, coordp, w0, b0, w1e)


def _make_gather(num_edges, width, chunk):
    info = plsc.get_sparse_core_info()
    nc, ns = info.num_cores, info.num_subcores
    nw = nc * ns
    per_w = num_edges // nw
    steps = per_w // chunk
    mesh = plsc.VectorSubcoreMesh(core_axis_name="c", subcore_axis_name="s")

    @functools.partial(
        pl.kernel,
        out_type=(
            jax.ShapeDtypeStruct((num_edges, width), jnp.float32),
            jax.ShapeDtypeStruct((num_edges, 16), jnp.float32),
        ),
        mesh=mesh,
        scratch_types=[
            pltpu.VMEM((chunk,), jnp.int32),
            pltpu.VMEM((chunk,), jnp.int32),
            pltpu.VMEM((chunk, width), jnp.float32),
            pltpu.VMEM((chunk, 16), jnp.float32),
            pltpu.SemaphoreType.DMA,
            pltpu.SemaphoreType.DMA,
        ],
        compiler_params=pltpu.CompilerParams(use_tc_tiling_on_sc=False),
    )
    def gather_kernel(table_hbm, coordp_hbm, idxj_hbm, idxi_hbm,
                      gj_hbm, ci_hbm, idxj_v, idxi_v, rows_v, ci_v,
                      sem_j, sem_i):
        wid = lax.axis_index("s") * nc + lax.axis_index("c")
        base_w = wid * per_w

        def body(i, _):
            base = base_w + i * chunk
            pltpu.sync_copy(idxj_hbm.at[pl.ds(base, chunk)], idxj_v)
            pltpu.sync_copy(idxi_hbm.at[pl.ds(base, chunk)], idxi_v)
            cp_j = pltpu.make_async_copy(table_hbm.at[idxj_v], rows_v, sem_j)
            cp_i = pltpu.make_async_copy(coordp_hbm.at[idxi_v], ci_v, sem_i)
            cp_j.start()
            cp_i.start()
            cp_j.wait()
            cp_i.wait()
            pltpu.sync_copy(rows_v, gj_hbm.at[pl.ds(base, chunk)])
            pltpu.sync_copy(ci_v, ci_hbm.at[pl.ds(base, chunk)])
            return _

        lax.fori_loop(0, steps, body, 0)

    return gather_kernel


def _edge_body(gj_ref, ci_ref, wr0i_ref, wr1i_ref, wr0_ref, wr1_ref,
               br0_ref, br1_ref, p16_ref, q_ref, q2_ref,
               out0_ref, out1_ref):
    c_out = wr0_ref.shape[1]
    gj = gj_ref[...]
    g0 = gj[:, 0:c_out]
    g1i = gj[:, c_out:4 * c_out]
    cj = gj[:, 4 * c_out:4 * c_out + 16]
    rij16 = cj - ci_ref[...]
    d2 = jnp.sum(rij16 * rij16, axis=1, keepdims=True) + 1e-6
    d = jnp.sqrt(d2)
    rinv = 1.0 / d
    be = gj.shape[0]
    centers = lax.broadcasted_iota(jnp.int32, (be, 16), 1).astype(
        jnp.float32) * (CUTOFF / 15.0)
    delta = d - centers
    rbf = jnp.exp(-GAMMA * delta * delta)
    fn0i = jnp.dot(rbf, wr0i_ref[...], preferred_element_type=jnp.float32)
    fn0i = fn0i + br0i_bcast(br0_ref)
    fn1i = jnp.dot(rbf, wr1i_ref[...], preferred_element_type=jnp.float32)
    fn1i = fn1i + br0i_bcast(br1_ref)
    fn0 = jnp.dot(rbf, wr0_ref[...], preferred_element_type=jnp.float32)
    fn0 = fn0 + br0_col(br0_ref, c_out)
    fn1 = jnp.dot(rbf, wr1_ref[...], preferred_element_type=jnp.float32)
    fn1 = fn1 + br0_col(br1_ref, c_out)
    ui = jnp.dot(rij16, p16_ref[...], preferred_element_type=jnp.float32) * rinv
    x0i = jnp.dot(g0, q_ref[...], preferred_element_type=jnp.float32)
    t = g1i * ui * fn1i
    out1_ref[...] = x0i * ui * fn1i + g1i * fn0i
    out0_ref[...] = g0 * fn0 + jnp.dot(t, q2_ref[...],
                                       preferred_element_type=jnp.float32)


def br0i_bcast(br_ref):
    c_out = br_ref.shape[1] // 4
    return br_ref[:, 0:3 * c_out]


def br0_col(br_ref, c_out):
    return br_ref[:, 3 * c_out:4 * c_out]


def _edge_stage(gj, ci, wr0i, wr1i, wr0, wr1, br0p, br1p, p16, q, q2, be):
    e = gj.shape[0]
    width = gj.shape[1]
    c_out = wr0.shape[1]
    grid = e // be
    full = lambda a: pl.BlockSpec(a.shape, lambda i: (0, 0))
    return pl.pallas_call(
        _edge_body,
        grid=(grid,),
        in_specs=[
            pl.BlockSpec((be, width), lambda i: (i, 0)),
            pl.BlockSpec((be, 16), lambda i: (i, 0)),
            full(wr0i), full(wr1i), full(wr0), full(wr1),
            full(br0p), full(br1p), full(p16), full(q), full(q2),
        ],
        out_specs=[
            pl.BlockSpec((be, c_out), lambda i: (i, 0)),
            pl.BlockSpec((be, 3 * c_out), lambda i: (i, 0)),
        ],
        out_shape=[
            jax.ShapeDtypeStruct((e, c_out), jnp.float32),
            jax.ShapeDtypeStruct((e, 3 * c_out), jnp.float32),
        ],
    )(gj, ci, wr0i, wr1i, wr0, wr1, br0p, br1p, p16, q, q2)


def kernel(node_0, node_1, coord, idx_i, idx_j, W0, b0, W1, Wr0, br0, Wr1,
           br1):
    n, c_in = node_0.shape
    c_out = W0.shape[1]
    e = idx_i.shape[0]

    n1f = node_1.reshape(n, 3 * c_in)
    coordp = jnp.pad(coord, ((0, 0), (0, 13)))
    eye3 = jnp.eye(3, dtype=jnp.float32)
    w1e = (W1[:, None, :, None] * eye3[None, :, None, :]).reshape(
        3 * c_in, 3 * c_out)
    wr0i = jnp.repeat(Wr0, 3, axis=1)
    wr1i = jnp.repeat(Wr1, 3, axis=1)
    br0p = jnp.concatenate([jnp.repeat(br0, 3), br0])[None, :]
    br1p = jnp.concatenate([jnp.repeat(br1, 3), br1])[None, :]
    p16 = jnp.concatenate(
        [jnp.tile(eye3, (1, c_out)), jnp.zeros((13, 3 * c_out))], axis=0)
    q = jnp.repeat(jnp.eye(c_out, dtype=jnp.float32), 3, axis=1)
    q2 = q.T
    idxj32 = idx_j.astype(jnp.int32)
    idxi32 = idx_i.astype(jnp.int32)

    table = _build_table(node_0, n1f, coordp, W0, b0[None, :], w1e, bn=1000)

    gather_kernel = _make_gather(e, 4 * c_out + 16, chunk=40)
    gj, ci = gather_kernel(table, coordp, idxj32, idxi32)

    out0, out1f = _edge_stage(gj, ci, wr0i, wr1i, Wr0, Wr1, br0p, br1p,
                              p16, q, q2, be=2000)
    return out0, out1f.reshape(e, c_out, 3)


# speedup vs baseline: 3.1060x; 2.2385x over previous
import functools
import jax
import jax.numpy as jnp
from jax import lax
from jax.experimental import pallas as pl
from jax.experimental.pallas import tpu as pltpu
from jax.experimental.pallas import tpu_sc as plsc

CUTOFF = 5.0
GAMMA = 10.0


def _table_body(n0_ref, n1_ref, w0_ref, b0_ref, w1_ref, t_ref):
    x0 = jnp.dot(n0_ref[...], w0_ref[...], preferred_element_type=jnp.float32)
    parts = [x0 + b0_ref[...]]
    for a in range(3):
        parts.append(jnp.dot(n1_ref[a], w1_ref[...],
                             preferred_element_type=jnp.float32))
    t_ref[...] = jnp.concatenate(parts, axis=1)


def _build_table(n0, n1p, w0, b0, w1, bn):
    n = n0.shape[0]
    c_out = w0.shape[1]
    width = 4 * c_out
    grid = n // bn
    return pl.pallas_call(
        _table_body,
        grid=(grid,),
        in_specs=[
            pl.BlockSpec((bn, n0.shape[1]), lambda i: (i, 0)),
            pl.BlockSpec((3, bn, n0.shape[1]), lambda i: (0, i, 0)),
            pl.BlockSpec(w0.shape, lambda i: (0, 0)),
            pl.BlockSpec((1, c_out), lambda i: (0, 0)),
            pl.BlockSpec(w1.shape, lambda i: (0, 0)),
        ],
        out_specs=pl.BlockSpec((bn, width), lambda i: (i, 0)),
        out_shape=jax.ShapeDtypeStruct((n, width), jnp.float32),
    )(n0, n1p, w0, b0, w1)


def _make_feature_gather(num_edges, width, chunk):
    info = plsc.get_sparse_core_info()
    nc, ns = info.num_cores, info.num_subcores
    nw = nc * ns
    per_w = num_edges // nw
    steps = per_w // chunk
    mesh = plsc.VectorSubcoreMesh(core_axis_name="c", subcore_axis_name="s")

    @functools.partial(
        pl.kernel,
        out_type=jax.ShapeDtypeStruct((num_edges, width), jnp.float32),
        mesh=mesh,
        scratch_types=[
            pltpu.VMEM((chunk,), jnp.int32),
            pltpu.VMEM((chunk, width), jnp.float32),
            pltpu.SemaphoreType.DMA,
        ],
    )
    def feature_gather(table_hbm, idxj_hbm, gj_hbm, idxj_v, rows_v, sem):
        wid = lax.axis_index("s") * nc + lax.axis_index("c")
        base_w = wid * per_w

        def body(i, carry):
            base = base_w + i * chunk
            pltpu.sync_copy(idxj_hbm.at[pl.ds(base, chunk)], idxj_v)
            cp = pltpu.make_async_copy(table_hbm.at[idxj_v], rows_v, sem)
            cp.start()
            cp.wait()
            pltpu.sync_copy(rows_v, gj_hbm.at[pl.ds(base, chunk)])
            return carry

        lax.fori_loop(0, steps, body, 0)

    return feature_gather


def _make_rij_gather(num_edges, chunk):
    info = plsc.get_sparse_core_info()
    nc, ns = info.num_cores, info.num_subcores
    nw = nc * ns
    per_w = num_edges // nw
    steps = per_w // chunk
    mesh = plsc.VectorSubcoreMesh(core_axis_name="c", subcore_axis_name="s")

    @functools.partial(
        pl.kernel,
        out_type=jax.ShapeDtypeStruct((16, num_edges), jnp.float32),
        mesh=mesh,
        scratch_types=[
            pltpu.VMEM((chunk,), jnp.int32),
            pltpu.VMEM((chunk,), jnp.int32),
            pltpu.VMEM((chunk, 16), jnp.float32),
            pltpu.VMEM((chunk, 16), jnp.float32),
            pltpu.VMEM((16 * num_edges // nw,), jnp.float32),
            pltpu.SemaphoreType.DMA,
            pltpu.SemaphoreType.DMA,
        ],
        compiler_params=pltpu.CompilerParams(use_tc_tiling_on_sc=False,
                                             needs_layout_passes=False),
    )
    def rij_gather(coordp_hbm, idxi_hbm, idxj_hbm, rijt_hbm,
                   idxi_v, idxj_v, ci_v, cj_v, rijt_v, sem_i, sem_j):
        wid = lax.axis_index("s") * nc + lax.axis_index("c")
        base_w = wid * per_w
        lane = lax.iota(jnp.int32, 16)

        def body(i, carry):
            base = i * chunk
            pltpu.sync_copy(idxi_hbm.at[pl.ds(base_w + base, chunk)], idxi_v)
            pltpu.sync_copy(idxj_hbm.at[pl.ds(base_w + base, chunk)], idxj_v)
            cp_i = pltpu.make_async_copy(coordp_hbm.at[idxi_v], ci_v, sem_i)
            cp_j = pltpu.make_async_copy(coordp_hbm.at[idxj_v], cj_v, sem_j)
            cp_i.start()
            cp_j.start()
            cp_i.wait()
            cp_j.wait()

            def inner(e, c2):
                rij = cj_v[e, :] - ci_v[e, :]
                plsc.store_scatter(rijt_v, [lane * per_w + (base + e)], rij)
                return c2

            lax.fori_loop(0, chunk, inner, 0)
            return carry

        lax.fori_loop(0, steps, body, 0)
        for k in range(16):
            pltpu.sync_copy(rijt_v.at[pl.ds(k * per_w, per_w)],
                            rijt_hbm.at[k, pl.ds(base_w, per_w)])

    return rij_gather


def _dot00(a, b):
    return lax.dot_general(a, b, (((0,), (0,)), ((), ())),
                           preferred_element_type=jnp.float32)


def _edge_body(gj_ref, rijt_ref, wr0p_ref, wr1p_ref, wr0_ref,
               brp0_ref, brp1_ref, br0_ref, p16p_ref, qp2_ref,
               out0_ref, out1_ref):
    c_out = wr0_ref.shape[1]
    gjt = jnp.transpose(gj_ref[...])
    g0t = gjt[0:c_out, :]
    g1pt = gjt[c_out:4 * c_out, :]
    rijt = rijt_ref[...]
    be = rijt.shape[1]
    d2 = jnp.sum(rijt * rijt, axis=0, keepdims=True) + 1e-6
    d = jnp.sqrt(d2)
    rinv = 1.0 / d
    centers = lax.broadcasted_iota(jnp.int32, (16, be), 0).astype(
        jnp.float32) * (CUTOFF / 15.0)
    delta = d - centers
    rbf = jnp.exp(-GAMMA * delta * delta)
    fn0p = _dot00(wr0p_ref[...], rbf) + brp0_ref[...]
    fn1p = _dot00(wr1p_ref[...], rbf) + brp1_ref[...]
    fn0 = _dot00(wr0_ref[...], rbf) + br0_ref[...]
    uip = _dot00(p16p_ref[...], rijt) * rinv
    x0ip = jnp.dot(qp2_ref[...], g0t, preferred_element_type=jnp.float32)
    t = g1pt * uip * fn1p
    out1_ref[...] = x0ip * uip * fn1p + g1pt * fn0p
    out0_ref[...] = g0t * fn0 + _dot00(qp2_ref[...], t)


def _edge_stage(gj, rijt, wr0p, wr1p, wr0, brp0, brp1, br0c, p16p, qp2, be):
    e = gj.shape[0]
    width = gj.shape[1]
    c_out = wr0.shape[1]
    grid = e // be
    full = lambda a: pl.BlockSpec(a.shape, lambda i: (0, 0))
    return pl.pallas_call(
        _edge_body,
        grid=(grid,),
        in_specs=[
            pl.BlockSpec((be, width), lambda i: (i, 0)),
            pl.BlockSpec((16, be), lambda i: (0, i)),
            full(wr0p), full(wr1p), full(wr0),
            full(brp0), full(brp1), full(br0c), full(p16p), full(qp2),
        ],
        out_specs=[
            pl.BlockSpec((c_out, be), lambda i: (0, i)),
            pl.BlockSpec((3 * c_out, be), lambda i: (0, i)),
        ],
        out_shape=[
            jax.ShapeDtypeStruct((c_out, e), jnp.float32),
            jax.ShapeDtypeStruct((3 * c_out, e), jnp.float32),
        ],
    )(gj, rijt, wr0p, wr1p, wr0, brp0, brp1, br0c, p16p, qp2)


def kernel(node_0, node_1, coord, idx_i, idx_j, W0, b0, W1, Wr0, br0, Wr1,
           br1):
    n, c_in = node_0.shape
    c_out = W0.shape[1]
    e = idx_i.shape[0]

    n1p = jnp.transpose(node_1, (2, 0, 1))
    coordp = jnp.pad(coord, ((0, 0), (0, 13)))
    eye3 = jnp.eye(3, dtype=jnp.float32)
    wr0p = jnp.tile(Wr0, (1, 3))
    wr1p = jnp.tile(Wr1, (1, 3))
    brp0 = jnp.tile(br0, 3)[:, None]
    brp1 = jnp.tile(br1, 3)[:, None]
    br0c = br0[:, None]
    p16p = jnp.concatenate(
        [jnp.repeat(eye3, c_out, axis=1), jnp.zeros((13, 3 * c_out))], axis=0)
    qp2 = jnp.tile(jnp.eye(c_out, dtype=jnp.float32), (3, 1))
    idxj32 = idx_j.astype(jnp.int32)
    idxi32 = idx_i.astype(jnp.int32)

    table = _build_table(node_0, n1p, W0, b0[None, :], W1, bn=1000)

    gj = _make_feature_gather(e, 4 * c_out, chunk=40)(table, idxj32)
    rijt = _make_rij_gather(e, chunk=40)(coordp, idxi32, idxj32)

    out0t, out1t = _edge_stage(gj, rijt, wr0p, wr1p, Wr0, brp0, brp1, br0c,
                               p16p, qp2, be=3200)
    out0 = out0t.T
    out1 = out1t.reshape(3, c_out, e).transpose(2, 1, 0)
    return out0, out1


# speedup vs baseline: 5.0976x; 1.6412x over previous
import functools
import jax
import jax.numpy as jnp
from jax import lax
from jax.experimental import pallas as pl
from jax.experimental.pallas import tpu as pltpu
from jax.experimental.pallas import tpu_sc as plsc

CUTOFF = 5.0
GAMMA = 10.0


def _table_body(n0_ref, n1_ref, w0_ref, b0_ref, w1_ref, t_ref):
    x0 = jnp.dot(n0_ref[...], w0_ref[...], preferred_element_type=jnp.float32)
    parts = [x0 + b0_ref[...]]
    for a in range(3):
        parts.append(jnp.dot(n1_ref[a], w1_ref[...],
                             preferred_element_type=jnp.float32))
    t_ref[...] = jnp.concatenate(parts, axis=1)


def _build_table(n0, n1p, w0, b0, w1, bn):
    n = n0.shape[0]
    c_out = w0.shape[1]
    width = 4 * c_out
    grid = n // bn
    return pl.pallas_call(
        _table_body,
        grid=(grid,),
        in_specs=[
            pl.BlockSpec((bn, n0.shape[1]), lambda i: (i, 0)),
            pl.BlockSpec((3, bn, n0.shape[1]), lambda i: (0, i, 0)),
            pl.BlockSpec(w0.shape, lambda i: (0, 0)),
            pl.BlockSpec((1, c_out), lambda i: (0, 0)),
            pl.BlockSpec(w1.shape, lambda i: (0, 0)),
        ],
        out_specs=pl.BlockSpec((bn, width), lambda i: (i, 0)),
        out_shape=jax.ShapeDtypeStruct((n, width), jnp.float32),
    )(n0, n1p, w0, b0, w1)


CHUNK = 128


def _make_edge_gather(num_edges, width):
    info = plsc.get_sparse_core_info()
    nc, ns = info.num_cores, info.num_subcores
    nw = nc * ns
    n_tiles = num_edges // CHUNK
    base_steps = n_tiles // nw
    extra = n_tiles - base_steps * nw
    mesh = plsc.VectorSubcoreMesh(core_axis_name="c", subcore_axis_name="s")

    @functools.partial(
        pl.kernel,
        out_type=(
            jax.ShapeDtypeStruct((num_edges, width), jnp.float32),
            jax.ShapeDtypeStruct((8, num_edges), jnp.float32),
        ),
        mesh=mesh,
        scratch_types=[
            pltpu.VMEM((CHUNK,), jnp.int32),
            pltpu.VMEM((CHUNK,), jnp.int32),
            pltpu.VMEM((CHUNK, width), jnp.float32),
            pltpu.VMEM((CHUNK,), jnp.float32),
            pltpu.VMEM((CHUNK,), jnp.float32),
            pltpu.VMEM((CHUNK,), jnp.float32),
            pltpu.VMEM((CHUNK,), jnp.float32),
            pltpu.VMEM((CHUNK,), jnp.float32),
            pltpu.VMEM((CHUNK,), jnp.float32),
            pltpu.VMEM((8, CHUNK), jnp.float32),
            pltpu.SemaphoreType.DMA,
        ],
        compiler_params=pltpu.CompilerParams(needs_layout_passes=False),
    )
    def edge_gather(table_hbm, cx_hbm, cy_hbm, cz_hbm, idxi_hbm, idxj_hbm,
                    gj_hbm, rijt_hbm,
                    idxi_v, idxj_v, rows_v, cxi_v, cyi_v, czi_v,
                    cxj_v, cyj_v, czj_v, tb_v, sem):
        wid = lax.axis_index("s") * nc + lax.axis_index("c")
        zero16 = jnp.zeros((16,), jnp.float32)
        for r in range(3, 8):
            for g in range(CHUNK // 16):
                tb_v[r, pl.ds(g * 16, 16)] = zero16

        def do_tile(t):
            base = t * CHUNK
            pltpu.sync_copy(idxj_hbm.at[pl.ds(base, CHUNK)], idxj_v)
            pltpu.sync_copy(idxi_hbm.at[pl.ds(base, CHUNK)], idxi_v)
            cps = [
                pltpu.make_async_copy(table_hbm.at[idxj_v], rows_v, sem),
                pltpu.make_async_copy(cx_hbm.at[idxi_v], cxi_v, sem),
                pltpu.make_async_copy(cy_hbm.at[idxi_v], cyi_v, sem),
                pltpu.make_async_copy(cz_hbm.at[idxi_v], czi_v, sem),
                pltpu.make_async_copy(cx_hbm.at[idxj_v], cxj_v, sem),
                pltpu.make_async_copy(cy_hbm.at[idxj_v], cyj_v, sem),
                pltpu.make_async_copy(cz_hbm.at[idxj_v], czj_v, sem),
            ]
            for cp in cps:
                cp.start()
            for cp in cps:
                cp.wait()
            for g in range(CHUNK // 16):
                s = pl.ds(g * 16, 16)
                tb_v[0, s] = cxj_v[s] - cxi_v[s]
                tb_v[1, s] = cyj_v[s] - cyi_v[s]
                tb_v[2, s] = czj_v[s] - czi_v[s]
            pltpu.sync_copy(rows_v, gj_hbm.at[pl.ds(base, CHUNK)])
            pltpu.sync_copy(tb_v, rijt_hbm.at[:, pl.ds(base, CHUNK)])

        def body(i, carry):
            do_tile(wid + i * nw)
            return carry

        lax.fori_loop(0, base_steps, body, 0)

        @pl.when(wid < extra)
        def _tail():
            do_tile(wid + base_steps * nw)

    return edge_gather


def _dot00(a, b):
    return lax.dot_general(a, b, (((0,), (0,)), ((), ())),
                           preferred_element_type=jnp.float32)


def _edge_body(gj_ref, rijt_ref, wr0p_ref, wr1p_ref, wr0_ref,
               brp0_ref, brp1_ref, br0_ref, p8p_ref, qp2_ref,
               out0_ref, out1_ref):
    c_out = wr0_ref.shape[1]
    gjt = jnp.transpose(gj_ref[...])
    g0t = gjt[0:c_out, :]
    g1pt = gjt[c_out:4 * c_out, :]
    rijt = rijt_ref[...]
    be = rijt.shape[1]
    d2 = jnp.sum(rijt * rijt, axis=0, keepdims=True) + 1e-6
    d = jnp.sqrt(d2)
    rinv = 1.0 / d
    centers = lax.broadcasted_iota(jnp.int32, (16, be), 0).astype(
        jnp.float32) * (CUTOFF / 15.0)
    delta = d - centers
    rbf = jnp.exp(-GAMMA * delta * delta)
    fn0p = _dot00(wr0p_ref[...], rbf) + brp0_ref[...]
    fn1p = _dot00(wr1p_ref[...], rbf) + brp1_ref[...]
    fn0 = _dot00(wr0_ref[...], rbf) + br0_ref[...]
    uip = _dot00(p8p_ref[...], rijt) * rinv
    x0ip = jnp.dot(qp2_ref[...], g0t, preferred_element_type=jnp.float32)
    t = g1pt * uip * fn1p
    out1_ref[...] = x0ip * uip * fn1p + g1pt * fn0p
    out0_ref[...] = g0t * fn0 + _dot00(qp2_ref[...], t)


def _edge_stage(gj, rijt, wr0p, wr1p, wr0, brp0, brp1, br0c, p8p, qp2, be):
    e = gj.shape[0]
    width = gj.shape[1]
    c_out = wr0.shape[1]
    grid = e // be
    full = lambda a: pl.BlockSpec(a.shape, lambda i: (0, 0))
    return pl.pallas_call(
        _edge_body,
        grid=(grid,),
        in_specs=[
            pl.BlockSpec((be, width), lambda i: (i, 0)),
            pl.BlockSpec((8, be), lambda i: (0, i)),
            full(wr0p), full(wr1p), full(wr0),
            full(brp0), full(brp1), full(br0c), full(p8p), full(qp2),
        ],
        out_specs=[
            pl.BlockSpec((c_out, be), lambda i: (0, i)),
            pl.BlockSpec((3 * c_out, be), lambda i: (0, i)),
        ],
        out_shape=[
            jax.ShapeDtypeStruct((c_out, e), jnp.float32),
            jax.ShapeDtypeStruct((3 * c_out, e), jnp.float32),
        ],
    )(gj, rijt, wr0p, wr1p, wr0, brp0, brp1, br0c, p8p, qp2)


def kernel(node_0, node_1, coord, idx_i, idx_j, W0, b0, W1, Wr0, br0, Wr1,
           br1):
    n, c_in = node_0.shape
    c_out = W0.shape[1]
    e = idx_i.shape[0]

    n1p = jnp.transpose(node_1, (2, 0, 1))
    coordt = jnp.transpose(coord)
    eye3 = jnp.eye(3, dtype=jnp.float32)
    wr0p = jnp.tile(Wr0, (1, 3))
    wr1p = jnp.tile(Wr1, (1, 3))
    brp0 = jnp.tile(br0, 3)[:, None]
    brp1 = jnp.tile(br1, 3)[:, None]
    br0c = br0[:, None]
    p8p = jnp.concatenate(
        [jnp.repeat(eye3, c_out, axis=1), jnp.zeros((5, 3 * c_out))], axis=0)
    qp2 = jnp.tile(jnp.eye(c_out, dtype=jnp.float32), (3, 1))
    idxj32 = idx_j.astype(jnp.int32)
    idxi32 = idx_i.astype(jnp.int32)

    table = _build_table(node_0, n1p, W0, b0[None, :], W1, bn=1000)

    gj, rijt = _make_edge_gather(e, 4 * c_out)(
        table, coordt[0], coordt[1], coordt[2], idxi32, idxj32)

    out0t, out1t = _edge_stage(gj, rijt, wr0p, wr1p, Wr0, brp0, brp1, br0c,
                               p8p, qp2, be=3200)
    out0 = out0t.T
    out1 = out1t.reshape(3, c_out, e).transpose(2, 1, 0)
    return out0, out1


# speedup vs baseline: 5.7985x; 1.1375x over previous
import functools
import jax
import jax.numpy as jnp
from jax import lax
from jax.experimental import pallas as pl
from jax.experimental.pallas import tpu as pltpu
from jax.experimental.pallas import tpu_sc as plsc

CUTOFF = 5.0
GAMMA = 10.0


def _table_body(n0_ref, n1_ref, w0_ref, b0_ref, w1_ref, t_ref):
    x0 = jnp.dot(n0_ref[...], w0_ref[...], preferred_element_type=jnp.float32)
    parts = [x0 + b0_ref[...]]
    for a in range(3):
        parts.append(jnp.dot(n1_ref[a], w1_ref[...],
                             preferred_element_type=jnp.float32))
    t_ref[...] = jnp.concatenate(parts, axis=1)


def _build_table(n0, n1p, w0, b0, w1, bn):
    n = n0.shape[0]
    c_out = w0.shape[1]
    width = 4 * c_out
    grid = n // bn
    return pl.pallas_call(
        _table_body,
        grid=(grid,),
        in_specs=[
            pl.BlockSpec((bn, n0.shape[1]), lambda i: (i, 0)),
            pl.BlockSpec((3, bn, n0.shape[1]), lambda i: (0, i, 0)),
            pl.BlockSpec(w0.shape, lambda i: (0, 0)),
            pl.BlockSpec((1, c_out), lambda i: (0, 0)),
            pl.BlockSpec(w1.shape, lambda i: (0, 0)),
        ],
        out_specs=pl.BlockSpec((bn, width), lambda i: (i, 0)),
        out_shape=jax.ShapeDtypeStruct((n, width), jnp.float32),
    )(n0, n1p, w0, b0, w1)


CHUNK = 128


def _make_edge_gather(num_edges, width):
    info = plsc.get_sparse_core_info()
    nc, ns = info.num_cores, info.num_subcores
    nw = nc * ns
    n_tiles = num_edges // CHUNK
    base_steps = n_tiles // nw
    extra = n_tiles - base_steps * nw
    mesh = plsc.VectorSubcoreMesh(core_axis_name="c", subcore_axis_name="s")

    @functools.partial(
        pl.kernel,
        out_type=(
            jax.ShapeDtypeStruct((num_edges, width), jnp.float32),
            jax.ShapeDtypeStruct((8, num_edges), jnp.float32),
        ),
        mesh=mesh,
        scratch_types=[
            [pltpu.VMEM((CHUNK,), jnp.int32) for _ in range(2)],
            [pltpu.VMEM((CHUNK,), jnp.int32) for _ in range(2)],
            [pltpu.VMEM((CHUNK, width), jnp.float32) for _ in range(2)],
            [[pltpu.VMEM((CHUNK,), jnp.float32) for _ in range(6)]
             for _ in range(2)],
            [pltpu.VMEM((8, CHUNK), jnp.float32) for _ in range(2)],
            [pltpu.SemaphoreType.DMA for _ in range(2)],
        ],
        compiler_params=pltpu.CompilerParams(needs_layout_passes=False),
    )
    def edge_gather(table_hbm, cx_hbm, cy_hbm, cz_hbm, idxi_hbm, idxj_hbm,
                    gj_hbm, rijt_hbm,
                    idxi_v, idxj_v, rows_v, cvs, tb_v, sems):
        wid = lax.axis_index("s") * nc + lax.axis_index("c")
        zero16 = jnp.zeros((16,), jnp.float32)
        for b in range(2):
            for r in range(3, 8):
                for g in range(CHUNK // 16):
                    tb_v[b][r, pl.ds(g * 16, 16)] = zero16

        def copies(b):
            return [
                pltpu.make_async_copy(table_hbm.at[idxj_v[b]], rows_v[b],
                                      sems[b]),
                pltpu.make_async_copy(cx_hbm.at[idxi_v[b]], cvs[b][0],
                                      sems[b]),
                pltpu.make_async_copy(cy_hbm.at[idxi_v[b]], cvs[b][1],
                                      sems[b]),
                pltpu.make_async_copy(cz_hbm.at[idxi_v[b]], cvs[b][2],
                                      sems[b]),
                pltpu.make_async_copy(cx_hbm.at[idxj_v[b]], cvs[b][3],
                                      sems[b]),
                pltpu.make_async_copy(cy_hbm.at[idxj_v[b]], cvs[b][4],
                                      sems[b]),
                pltpu.make_async_copy(cz_hbm.at[idxj_v[b]], cvs[b][5],
                                      sems[b]),
            ]

        def start(t, b):
            base = t * CHUNK
            pltpu.sync_copy(idxj_hbm.at[pl.ds(base, CHUNK)], idxj_v[b])
            pltpu.sync_copy(idxi_hbm.at[pl.ds(base, CHUNK)], idxi_v[b])
            for cp in copies(b):
                cp.start()

        def finish(t, b):
            base = t * CHUNK
            for cp in copies(b):
                cp.wait()
            for g in range(CHUNK // 16):
                s = pl.ds(g * 16, 16)
                tb_v[b][0, s] = cvs[b][3][s] - cvs[b][0][s]
                tb_v[b][1, s] = cvs[b][4][s] - cvs[b][1][s]
                tb_v[b][2, s] = cvs[b][5][s] - cvs[b][2][s]
            pltpu.sync_copy(rows_v[b], gj_hbm.at[pl.ds(base, CHUNK)])
            pltpu.sync_copy(tb_v[b], rijt_hbm.at[:, pl.ds(base, CHUNK)])

        pairs = base_steps // 2
        rem = base_steps % 2
        tile = lambda i: wid + i * nw
        start(tile(0), 0)

        def pair_body(p, carry):
            i = 2 * p
            start(tile(i + 1), 1)
            finish(tile(i), 0)
            if rem == 1:
                start(tile(i + 2), 0)
            else:
                @pl.when(i + 2 < base_steps)
                def _s():
                    start(tile(i + 2), 0)
            finish(tile(i + 1), 1)
            return carry

        lax.fori_loop(0, pairs, pair_body, 0)
        if rem == 1:
            finish(tile(base_steps - 1), 0)

        @pl.when(wid < extra)
        def _tail():
            t = wid + base_steps * nw
            start(t, 0)
            finish(t, 0)

    return edge_gather


def _dot00(a, b):
    return lax.dot_general(a, b, (((0,), (0,)), ((), ())),
                           preferred_element_type=jnp.float32)


def _edge_body(gj_ref, rijt_ref, wr0p_ref, wr1p_ref, wr0_ref,
               brp0_ref, brp1_ref, br0_ref, p8p_ref, qp2_ref,
               out0_ref, out1_ref):
    c_out = wr0_ref.shape[1]
    gjt = jnp.transpose(gj_ref[...])
    g0t = gjt[0:c_out, :]
    g1pt = gjt[c_out:4 * c_out, :]
    rijt = rijt_ref[...]
    be = rijt.shape[1]
    d2 = jnp.sum(rijt * rijt, axis=0, keepdims=True) + 1e-6
    d = jnp.sqrt(d2)
    rinv = 1.0 / d
    centers = lax.broadcasted_iota(jnp.int32, (16, be), 0).astype(
        jnp.float32) * (CUTOFF / 15.0)
    delta = d - centers
    rbf = jnp.exp(-GAMMA * delta * delta)
    fn0p = _dot00(wr0p_ref[...], rbf) + brp0_ref[...]
    fn1p = _dot00(wr1p_ref[...], rbf) + brp1_ref[...]
    fn0 = _dot00(wr0_ref[...], rbf) + br0_ref[...]
    uip = _dot00(p8p_ref[...], rijt) * rinv
    x0ip = jnp.dot(qp2_ref[...], g0t, preferred_element_type=jnp.float32)
    t = g1pt * uip * fn1p
    out1_ref[...] = x0ip * uip * fn1p + g1pt * fn0p
    out0_ref[...] = g0t * fn0 + _dot00(qp2_ref[...], t)


def _edge_stage(gj, rijt, wr0p, wr1p, wr0, brp0, brp1, br0c, p8p, qp2, be):
    e = gj.shape[0]
    width = gj.shape[1]
    c_out = wr0.shape[1]
    grid = e // be
    full = lambda a: pl.BlockSpec(a.shape, lambda i: (0, 0))
    return pl.pallas_call(
        _edge_body,
        grid=(grid,),
        in_specs=[
            pl.BlockSpec((be, width), lambda i: (i, 0)),
            pl.BlockSpec((8, be), lambda i: (0, i)),
            full(wr0p), full(wr1p), full(wr0),
            full(brp0), full(brp1), full(br0c), full(p8p), full(qp2),
        ],
        out_specs=[
            pl.BlockSpec((c_out, be), lambda i: (0, i)),
            pl.BlockSpec((3 * c_out, be), lambda i: (0, i)),
        ],
        out_shape=[
            jax.ShapeDtypeStruct((c_out, e), jnp.float32),
            jax.ShapeDtypeStruct((3 * c_out, e), jnp.float32),
        ],
    )(gj, rijt, wr0p, wr1p, wr0, brp0, brp1, br0c, p8p, qp2)


def kernel(node_0, node_1, coord, idx_i, idx_j, W0, b0, W1, Wr0, br0, Wr1,
           br1):
    n, c_in = node_0.shape
    c_out = W0.shape[1]
    e = idx_i.shape[0]

    n1p = jnp.transpose(node_1, (2, 0, 1))
    coordt = jnp.transpose(coord)
    eye3 = jnp.eye(3, dtype=jnp.float32)
    wr0p = jnp.tile(Wr0, (1, 3))
    wr1p = jnp.tile(Wr1, (1, 3))
    brp0 = jnp.tile(br0, 3)[:, None]
    brp1 = jnp.tile(br1, 3)[:, None]
    br0c = br0[:, None]
    p8p = jnp.concatenate(
        [jnp.repeat(eye3, c_out, axis=1), jnp.zeros((5, 3 * c_out))], axis=0)
    qp2 = jnp.tile(jnp.eye(c_out, dtype=jnp.float32), (3, 1))
    idxj32 = idx_j.astype(jnp.int32)
    idxi32 = idx_i.astype(jnp.int32)

    table = _build_table(node_0, n1p, W0, b0[None, :], W1, bn=1000)

    gj, rijt = _make_edge_gather(e, 4 * c_out)(
        table, coordt[0], coordt[1], coordt[2], idxi32, idxj32)

    out0t, out1t = _edge_stage(gj, rijt, wr0p, wr1p, Wr0, brp0, brp1, br0c,
                               p8p, qp2, be=3200)
    out0 = out0t.T
    out1 = out1t.reshape(3, c_out, e).transpose(2, 1, 0)
    return out0, out1


# speedup vs baseline: 7.1266x; 1.2290x over previous
import functools
import jax
import jax.numpy as jnp
from jax import lax
from jax.experimental import pallas as pl
from jax.experimental.pallas import tpu as pltpu
from jax.experimental.pallas import tpu_sc as plsc

CUTOFF = 5.0
GAMMA = 10.0


def _table_body(n0_ref, n1_ref, w0_ref, b0_ref, w1_ref, t_ref):
    x0 = jnp.dot(n0_ref[...], w0_ref[...], preferred_element_type=jnp.float32)
    parts = [x0 + b0_ref[...]]
    for a in range(3):
        parts.append(jnp.dot(n1_ref[a], w1_ref[...],
                             preferred_element_type=jnp.float32))
    h = jnp.concatenate(parts, axis=1)
    half = h.shape[1] // 2
    hu = lax.bitcast_convert_type(h[:, :half], jnp.uint32)
    lu = lax.bitcast_convert_type(h[:, half:], jnp.uint32)
    packed = (((hu + 0x8000) & jnp.uint32(0xFFFF0000))
              | ((lu + 0x8000) >> 16))
    t_ref[...] = lax.bitcast_convert_type(packed, jnp.int32)


def _build_table(n0, n1p, w0, b0, w1, bn):
    n = n0.shape[0]
    c_out = w0.shape[1]
    width = 2 * c_out
    grid = n // bn
    return pl.pallas_call(
        _table_body,
        grid=(grid,),
        in_specs=[
            pl.BlockSpec((bn, n0.shape[1]), lambda i: (i, 0)),
            pl.BlockSpec((3, bn, n0.shape[1]), lambda i: (0, i, 0)),
            pl.BlockSpec(w0.shape, lambda i: (0, 0)),
            pl.BlockSpec((1, c_out), lambda i: (0, 0)),
            pl.BlockSpec(w1.shape, lambda i: (0, 0)),
        ],
        out_specs=pl.BlockSpec((bn, width), lambda i: (i, 0)),
        out_shape=jax.ShapeDtypeStruct((n, width), jnp.int32),
    )(n0, n1p, w0, b0, w1)


CHUNK = 128


def _make_edge_gather(num_edges, width):
    info = plsc.get_sparse_core_info()
    nc, ns = info.num_cores, info.num_subcores
    nw = nc * ns
    n_tiles = num_edges // CHUNK
    base_steps = n_tiles // nw
    extra = n_tiles - base_steps * nw
    mesh = plsc.VectorSubcoreMesh(core_axis_name="c", subcore_axis_name="s")

    @functools.partial(
        pl.kernel,
        out_type=(
            jax.ShapeDtypeStruct((num_edges, width), jnp.int32),
            jax.ShapeDtypeStruct((8, num_edges), jnp.float32),
        ),
        mesh=mesh,
        scratch_types=[
            [pltpu.VMEM((CHUNK,), jnp.int32) for _ in range(2)],
            [pltpu.VMEM((CHUNK,), jnp.int32) for _ in range(2)],
            [pltpu.VMEM((CHUNK, width), jnp.int32) for _ in range(2)],
            [[pltpu.VMEM((CHUNK,), jnp.float32) for _ in range(6)]
             for _ in range(2)],
            [pltpu.VMEM((8, CHUNK), jnp.float32) for _ in range(2)],
            [pltpu.SemaphoreType.DMA for _ in range(2)],
        ],
        compiler_params=pltpu.CompilerParams(needs_layout_passes=False),
    )
    def edge_gather(table_hbm, cx_hbm, cy_hbm, cz_hbm, idxi_hbm, idxj_hbm,
                    gj_hbm, rijt_hbm,
                    idxi_v, idxj_v, rows_v, cvs, tb_v, sems):
        wid = lax.axis_index("s") * nc + lax.axis_index("c")
        zero16 = jnp.zeros((16,), jnp.float32)
        for b in range(2):
            for r in range(3, 8):
                for g in range(CHUNK // 16):
                    tb_v[b][r, pl.ds(g * 16, 16)] = zero16

        def copies(b):
            return [
                pltpu.make_async_copy(table_hbm.at[idxj_v[b]], rows_v[b],
                                      sems[b]),
                pltpu.make_async_copy(cx_hbm.at[idxi_v[b]], cvs[b][0],
                                      sems[b]),
                pltpu.make_async_copy(cy_hbm.at[idxi_v[b]], cvs[b][1],
                                      sems[b]),
                pltpu.make_async_copy(cz_hbm.at[idxi_v[b]], cvs[b][2],
                                      sems[b]),
                pltpu.make_async_copy(cx_hbm.at[idxj_v[b]], cvs[b][3],
                                      sems[b]),
                pltpu.make_async_copy(cy_hbm.at[idxj_v[b]], cvs[b][4],
                                      sems[b]),
                pltpu.make_async_copy(cz_hbm.at[idxj_v[b]], cvs[b][5],
                                      sems[b]),
            ]

        def start(t, b):
            base = t * CHUNK
            pltpu.sync_copy(idxj_hbm.at[pl.ds(base, CHUNK)], idxj_v[b])
            pltpu.sync_copy(idxi_hbm.at[pl.ds(base, CHUNK)], idxi_v[b])
            for cp in copies(b):
                cp.start()

        def finish(t, b):
            base = t * CHUNK
            for cp in copies(b):
                cp.wait()
            for g in range(CHUNK // 16):
                s = pl.ds(g * 16, 16)
                tb_v[b][0, s] = cvs[b][3][s] - cvs[b][0][s]
                tb_v[b][1, s] = cvs[b][4][s] - cvs[b][1][s]
                tb_v[b][2, s] = cvs[b][5][s] - cvs[b][2][s]
            pltpu.sync_copy(rows_v[b], gj_hbm.at[pl.ds(base, CHUNK)])
            pltpu.sync_copy(tb_v[b], rijt_hbm.at[:, pl.ds(base, CHUNK)])

        pairs = base_steps // 2
        rem = base_steps % 2
        tile = lambda i: wid + i * nw
        start(tile(0), 0)

        def pair_body(p, carry):
            i = 2 * p
            start(tile(i + 1), 1)
            finish(tile(i), 0)
            if rem == 1:
                start(tile(i + 2), 0)
            else:
                @pl.when(i + 2 < base_steps)
                def _s():
                    start(tile(i + 2), 0)
            finish(tile(i + 1), 1)
            return carry

        lax.fori_loop(0, pairs, pair_body, 0)
        if rem == 1:
            finish(tile(base_steps - 1), 0)

        @pl.when(wid < extra)
        def _tail():
            t = wid + base_steps * nw
            start(t, 0)
            finish(t, 0)

    return edge_gather


def _dot00(a, b):
    return lax.dot_general(a, b, (((0,), (0,)), ((), ())),
                           preferred_element_type=jnp.float32)


def _edge_body(gj_ref, rijt_ref, wr0p_ref, wr1p_ref, wr0_ref,
               brp0_ref, brp1_ref, br0_ref, p8p_ref, qp2_ref,
               out0_ref, out1_ref):
    c_out = wr0_ref.shape[1]
    gt = lax.bitcast_convert_type(jnp.transpose(gj_ref[...]),
                                  jnp.uint32)
    hi = lax.bitcast_convert_type(gt & jnp.uint32(0xFFFF0000), jnp.float32)
    lo = lax.bitcast_convert_type(gt << 16, jnp.float32)
    gjt = jnp.concatenate([hi, lo], axis=0)
    g0t = gjt[0:c_out, :]
    g1pt = gjt[c_out:4 * c_out, :]
    rijt = rijt_ref[...]
    be = rijt.shape[1]
    d2 = jnp.sum(rijt * rijt, axis=0, keepdims=True) + 1e-6
    d = jnp.sqrt(d2)
    rinv = 1.0 / d
    centers = lax.broadcasted_iota(jnp.int32, (16, be), 0).astype(
        jnp.float32) * (CUTOFF / 15.0)
    delta = d - centers
    rbf = jnp.exp(-GAMMA * delta * delta)
    fn0p = _dot00(wr0p_ref[...], rbf) + brp0_ref[...]
    fn1p = _dot00(wr1p_ref[...], rbf) + brp1_ref[...]
    fn0 = _dot00(wr0_ref[...], rbf) + br0_ref[...]
    uip = _dot00(p8p_ref[...], rijt) * rinv
    x0ip = jnp.dot(qp2_ref[...], g0t, preferred_element_type=jnp.float32)
    t = g1pt * uip * fn1p
    out1_ref[...] = x0ip * uip * fn1p + g1pt * fn0p
    out0_ref[...] = g0t * fn0 + _dot00(qp2_ref[...], t)


def _edge_stage(gj, rijt, wr0p, wr1p, wr0, brp0, brp1, br0c, p8p, qp2, be):
    e = gj.shape[0]
    width = gj.shape[1]
    c_out = wr0.shape[1]
    grid = e // be
    full = lambda a: pl.BlockSpec(a.shape, lambda i: (0, 0))
    return pl.pallas_call(
        _edge_body,
        grid=(grid,),
        in_specs=[
            pl.BlockSpec((be, width), lambda i: (i, 0)),
            pl.BlockSpec((8, be), lambda i: (0, i)),
            full(wr0p), full(wr1p), full(wr0),
            full(brp0), full(brp1), full(br0c), full(p8p), full(qp2),
        ],
        out_specs=[
            pl.BlockSpec((c_out, be), lambda i: (0, i)),
            pl.BlockSpec((3 * c_out, be), lambda i: (0, i)),
        ],
        out_shape=[
            jax.ShapeDtypeStruct((c_out, e), jnp.float32),
            jax.ShapeDtypeStruct((3 * c_out, e), jnp.float32),
        ],
    )(gj, rijt, wr0p, wr1p, wr0, brp0, brp1, br0c, p8p, qp2)


def kernel(node_0, node_1, coord, idx_i, idx_j, W0, b0, W1, Wr0, br0, Wr1,
           br1):
    n, c_in = node_0.shape
    c_out = W0.shape[1]
    e = idx_i.shape[0]

    n1p = jnp.transpose(node_1, (2, 0, 1))
    coordt = jnp.transpose(coord)
    eye3 = jnp.eye(3, dtype=jnp.float32)
    wr0p = jnp.tile(Wr0, (1, 3))
    wr1p = jnp.tile(Wr1, (1, 3))
    brp0 = jnp.tile(br0, 3)[:, None]
    brp1 = jnp.tile(br1, 3)[:, None]
    br0c = br0[:, None]
    p8p = jnp.concatenate(
        [jnp.repeat(eye3, c_out, axis=1), jnp.zeros((5, 3 * c_out))], axis=0)
    qp2 = jnp.tile(jnp.eye(c_out, dtype=jnp.float32), (3, 1))
    idxj32 = idx_j.astype(jnp.int32)
    idxi32 = idx_i.astype(jnp.int32)

    table = _build_table(node_0, n1p, W0, b0[None, :], W1, bn=1000)

    gj, rijt = _make_edge_gather(e, 2 * c_out)(
        table, coordt[0], coordt[1], coordt[2], idxi32, idxj32)

    out0t, out1t = _edge_stage(gj, rijt, wr0p, wr1p, Wr0, brp0, brp1, br0c,
                               p8p, qp2, be=3200)
    out0 = out0t.T
    out1 = out1t.reshape(3, c_out, e).transpose(2, 1, 0)
    return out0, out1


# speedup vs baseline: 8.0548x; 1.1302x over previous
import functools
import jax
import jax.numpy as jnp
from jax import lax
from jax.experimental import pallas as pl
from jax.experimental.pallas import tpu as pltpu
from jax.experimental.pallas import tpu_sc as plsc

CUTOFF = 5.0
GAMMA = 10.0


def _table_body(n0_ref, n1_ref, w0_ref, b0_ref, w1_ref, t_ref):
    x0 = jnp.dot(n0_ref[...], w0_ref[...], preferred_element_type=jnp.float32)
    parts = [x0 + b0_ref[...]]
    for a in range(3):
        parts.append(jnp.dot(n1_ref[a], w1_ref[...],
                             preferred_element_type=jnp.float32))
    h = jnp.concatenate(parts, axis=1)
    half = h.shape[1] // 2
    hu = lax.bitcast_convert_type(h[:, :half], jnp.uint32)
    lu = lax.bitcast_convert_type(h[:, half:], jnp.uint32)
    packed = (((hu + 0x8000) & jnp.uint32(0xFFFF0000))
              | ((lu + 0x8000) >> 16))
    t_ref[...] = lax.bitcast_convert_type(packed, jnp.int32)


def _build_table(n0, n1p, w0, b0, w1, bn):
    n = n0.shape[0]
    c_out = w0.shape[1]
    width = 2 * c_out
    grid = n // bn
    return pl.pallas_call(
        _table_body,
        grid=(grid,),
        in_specs=[
            pl.BlockSpec((bn, n0.shape[1]), lambda i: (i, 0)),
            pl.BlockSpec((3, bn, n0.shape[1]), lambda i: (0, i, 0)),
            pl.BlockSpec(w0.shape, lambda i: (0, 0)),
            pl.BlockSpec((1, c_out), lambda i: (0, 0)),
            pl.BlockSpec(w1.shape, lambda i: (0, 0)),
        ],
        out_specs=pl.BlockSpec((bn, width), lambda i: (i, 0)),
        out_shape=jax.ShapeDtypeStruct((n, width), jnp.int32),
    )(n0, n1p, w0, b0, w1)


CHUNK = 128


def _make_edge_gather(num_edges, width):
    info = plsc.get_sparse_core_info()
    nc, ns = info.num_cores, info.num_subcores
    nw = nc * ns
    n_tiles = num_edges // CHUNK
    base_steps = n_tiles // nw
    extra = n_tiles - base_steps * nw
    mesh = plsc.VectorSubcoreMesh(core_axis_name="c", subcore_axis_name="s")

    @functools.partial(
        pl.kernel,
        out_type=(
            jax.ShapeDtypeStruct((num_edges, width), jnp.int32),
            jax.ShapeDtypeStruct((8, num_edges), jnp.float32),
        ),
        mesh=mesh,
        scratch_types=[
            [pltpu.VMEM((CHUNK,), jnp.int32) for _ in range(2)],
            [pltpu.VMEM((CHUNK,), jnp.int32) for _ in range(2)],
            [pltpu.VMEM((CHUNK, width), jnp.int32) for _ in range(2)],
            [[pltpu.VMEM((CHUNK,), jnp.float32) for _ in range(6)]
             for _ in range(2)],
            [pltpu.VMEM((8, CHUNK), jnp.float32) for _ in range(2)],
            [pltpu.SemaphoreType.DMA for _ in range(2)],
        ],
        compiler_params=pltpu.CompilerParams(needs_layout_passes=False),
    )
    def edge_gather(table_hbm, cx_hbm, cy_hbm, cz_hbm, idxi_hbm, idxj_hbm,
                    gj_hbm, rijt_hbm,
                    idxi_v, idxj_v, rows_v, cvs, tb_v, sems):
        wid = lax.axis_index("s") * nc + lax.axis_index("c")
        zero16 = jnp.zeros((16,), jnp.float32)
        for b in range(2):
            for r in range(3, 8):
                for g in range(CHUNK // 16):
                    tb_v[b][r, pl.ds(g * 16, 16)] = zero16

        def copies(b):
            return [
                pltpu.make_async_copy(table_hbm.at[idxj_v[b]], rows_v[b],
                                      sems[b]),
                pltpu.make_async_copy(cx_hbm.at[idxi_v[b]], cvs[b][0],
                                      sems[b]),
                pltpu.make_async_copy(cy_hbm.at[idxi_v[b]], cvs[b][1],
                                      sems[b]),
                pltpu.make_async_copy(cz_hbm.at[idxi_v[b]], cvs[b][2],
                                      sems[b]),
                pltpu.make_async_copy(cx_hbm.at[idxj_v[b]], cvs[b][3],
                                      sems[b]),
                pltpu.make_async_copy(cy_hbm.at[idxj_v[b]], cvs[b][4],
                                      sems[b]),
                pltpu.make_async_copy(cz_hbm.at[idxj_v[b]], cvs[b][5],
                                      sems[b]),
            ]

        def start(t, b):
            base = t * CHUNK
            pltpu.sync_copy(idxj_hbm.at[pl.ds(base, CHUNK)], idxj_v[b])
            pltpu.sync_copy(idxi_hbm.at[pl.ds(base, CHUNK)], idxi_v[b])
            for cp in copies(b):
                cp.start()

        def finish(t, b):
            base = t * CHUNK
            for cp in copies(b):
                cp.wait()
            for g in range(CHUNK // 16):
                s = pl.ds(g * 16, 16)
                tb_v[b][0, s] = cvs[b][3][s] - cvs[b][0][s]
                tb_v[b][1, s] = cvs[b][4][s] - cvs[b][1][s]
                tb_v[b][2, s] = cvs[b][5][s] - cvs[b][2][s]
            pltpu.sync_copy(rows_v[b], gj_hbm.at[pl.ds(base, CHUNK)])
            pltpu.sync_copy(tb_v[b], rijt_hbm.at[:, pl.ds(base, CHUNK)])

        pairs = base_steps // 2
        rem = base_steps % 2
        tile = lambda i: wid + i * nw
        start(tile(0), 0)

        def pair_body(p, carry):
            i = 2 * p
            start(tile(i + 1), 1)
            finish(tile(i), 0)
            if rem == 1:
                start(tile(i + 2), 0)
            else:
                @pl.when(i + 2 < base_steps)
                def _s():
                    start(tile(i + 2), 0)
            finish(tile(i + 1), 1)
            return carry

        lax.fori_loop(0, pairs, pair_body, 0)
        if rem == 1:
            finish(tile(base_steps - 1), 0)

        @pl.when(wid < extra)
        def _tail():
            t = wid + base_steps * nw
            start(t, 0)
            finish(t, 0)

    return edge_gather


def _dot00(a, b):
    return lax.dot_general(a, b, (((0,), (0,)), ((), ())),
                           preferred_element_type=jnp.float32)


def _edge_body(gj_ref, rijt_ref, wr0_ref, wr1_ref, br0_ref, br1_ref,
               out0_ref, out1_ref):
    c_out = wr0_ref.shape[1]
    gt = lax.bitcast_convert_type(jnp.transpose(gj_ref[...]),
                                  jnp.uint32)
    hi = lax.bitcast_convert_type(gt & jnp.uint32(0xFFFF0000), jnp.float32)
    lo = lax.bitcast_convert_type(gt << 16, jnp.float32)
    g0t = hi[0:c_out, :]
    g1 = (hi[c_out:2 * c_out, :],
          lo[0:c_out, :],
          lo[c_out:2 * c_out, :])
    rijt = rijt_ref[...]
    be = rijt.shape[1]
    d2 = jnp.sum(rijt * rijt, axis=0, keepdims=True) + 1e-6
    d = jnp.sqrt(d2)
    rinv = 1.0 / d
    centers = lax.broadcasted_iota(jnp.int32, (16, be), 0).astype(
        jnp.float32) * (CUTOFF / 15.0)
    delta = d - centers
    rbf = jnp.exp(-GAMMA * delta * delta)
    fn0 = _dot00(wr0_ref[...], rbf) + br0_ref[...]
    fn1 = _dot00(wr1_ref[...], rbf) + br1_ref[...]
    acc = None
    for a in range(3):
        u_a = rijt[a:a + 1, :] * rinv
        out1_ref[pl.ds(a * c_out, c_out), :] = (g0t * u_a * fn1
                                                + g1[a] * fn0)
        ga_ua = g1[a] * u_a
        acc = ga_ua if acc is None else acc + ga_ua
    out0_ref[...] = g0t * fn0 + acc * fn1


def _edge_stage(gj, rijt, wr0, wr1, br0c, br1c, be):
    e = gj.shape[0]
    width = gj.shape[1]
    c_out = wr0.shape[1]
    grid = e // be
    full = lambda a: pl.BlockSpec(a.shape, lambda i: (0, 0))
    return pl.pallas_call(
        _edge_body,
        grid=(grid,),
        in_specs=[
            pl.BlockSpec((be, width), lambda i: (i, 0)),
            pl.BlockSpec((8, be), lambda i: (0, i)),
            full(wr0), full(wr1), full(br0c), full(br1c),
        ],
        out_specs=[
            pl.BlockSpec((c_out, be), lambda i: (0, i)),
            pl.BlockSpec((3 * c_out, be), lambda i: (0, i)),
        ],
        out_shape=[
            jax.ShapeDtypeStruct((c_out, e), jnp.float32),
            jax.ShapeDtypeStruct((3 * c_out, e), jnp.float32),
        ],
    )(gj, rijt, wr0, wr1, br0c, br1c)


def kernel(node_0, node_1, coord, idx_i, idx_j, W0, b0, W1, Wr0, br0, Wr1,
           br1):
    n, c_in = node_0.shape
    c_out = W0.shape[1]
    e = idx_i.shape[0]

    n1p = jnp.transpose(node_1, (2, 0, 1))
    coordt = jnp.transpose(coord)
    br0c = br0[:, None]
    br1c = br1[:, None]
    idxj32 = idx_j.astype(jnp.int32)
    idxi32 = idx_i.astype(jnp.int32)

    table = _build_table(node_0, n1p, W0, b0[None, :], W1, bn=1000)

    gj, rijt = _make_edge_gather(e, 2 * c_out)(
        table, coordt[0], coordt[1], coordt[2], idxi32, idxj32)

    out0t, out1t = _edge_stage(gj, rijt, Wr0, Wr1, br0c, br1c, be=3200)
    out0 = out0t.T
    out1 = out1t.reshape(3, c_out, e).transpose(2, 1, 0)
    return out0, out1


# speedup vs baseline: 8.3307x; 1.0343x over previous
import functools
import jax
import jax.numpy as jnp
from jax import lax
from jax.experimental import pallas as pl
from jax.experimental.pallas import tpu as pltpu
from jax.experimental.pallas import tpu_sc as plsc

CUTOFF = 5.0
GAMMA = 10.0


def _table_body(n0_ref, n1_ref, w0_ref, b0_ref, w1_ref, t_ref):
    x0 = jnp.dot(n0_ref[...], w0_ref[...], preferred_element_type=jnp.float32)
    parts = [x0 + b0_ref[...]]
    for a in range(3):
        parts.append(jnp.dot(n1_ref[a], w1_ref[...],
                             preferred_element_type=jnp.float32))
    h = jnp.concatenate(parts, axis=1)
    half = h.shape[1] // 2
    hu = lax.bitcast_convert_type(h[:, :half], jnp.uint32)
    lu = lax.bitcast_convert_type(h[:, half:], jnp.uint32)
    packed = (((hu + 0x8000) & jnp.uint32(0xFFFF0000))
              | ((lu + 0x8000) >> 16))
    t_ref[...] = lax.bitcast_convert_type(packed, jnp.int32)


def _build_table(n0, n1p, w0, b0, w1, bn):
    n = n0.shape[0]
    c_out = w0.shape[1]
    width = 2 * c_out
    grid = n // bn
    return pl.pallas_call(
        _table_body,
        grid=(grid,),
        in_specs=[
            pl.BlockSpec((bn, n0.shape[1]), lambda i: (i, 0)),
            pl.BlockSpec((3, bn, n0.shape[1]), lambda i: (0, i, 0)),
            pl.BlockSpec(w0.shape, lambda i: (0, 0)),
            pl.BlockSpec((1, c_out), lambda i: (0, 0)),
            pl.BlockSpec(w1.shape, lambda i: (0, 0)),
        ],
        out_specs=pl.BlockSpec((bn, width), lambda i: (i, 0)),
        out_shape=jax.ShapeDtypeStruct((n, width), jnp.int32),
    )(n0, n1p, w0, b0, w1)


CHUNK = 128


def _make_edge_gather(num_edges, width):
    info = plsc.get_sparse_core_info()
    nc, ns = info.num_cores, info.num_subcores
    nw = nc * ns
    n_tiles = num_edges // CHUNK
    base_steps = n_tiles // nw
    extra = n_tiles - base_steps * nw
    mesh = plsc.VectorSubcoreMesh(core_axis_name="c", subcore_axis_name="s")

    @functools.partial(
        pl.kernel,
        out_type=(
            jax.ShapeDtypeStruct((num_edges, width), jnp.int32),
            jax.ShapeDtypeStruct((8, num_edges), jnp.float32),
        ),
        mesh=mesh,
        scratch_types=[
            [pltpu.VMEM((CHUNK,), jnp.int32) for _ in range(2)],
            [pltpu.VMEM((CHUNK,), jnp.int32) for _ in range(2)],
            [pltpu.VMEM((CHUNK, width), jnp.int32) for _ in range(2)],
            [[pltpu.VMEM((CHUNK,), jnp.float32) for _ in range(6)]
             for _ in range(2)],
            [pltpu.VMEM((8, CHUNK), jnp.float32) for _ in range(2)],
            [pltpu.SemaphoreType.DMA for _ in range(2)],
        ],
        compiler_params=pltpu.CompilerParams(needs_layout_passes=False),
    )
    def edge_gather(table_hbm, cx_hbm, cy_hbm, cz_hbm, idxi_hbm, idxj_hbm,
                    gj_hbm, rijt_hbm,
                    idxi_v, idxj_v, rows_v, cvs, tb_v, sems):
        wid = lax.axis_index("s") * nc + lax.axis_index("c")
        zero16 = jnp.zeros((16,), jnp.float32)
        for b in range(2):
            for r in range(3, 8):
                for g in range(CHUNK // 16):
                    tb_v[b][r, pl.ds(g * 16, 16)] = zero16

        def copies(b):
            return [
                pltpu.make_async_copy(table_hbm.at[idxj_v[b]], rows_v[b],
                                      sems[b]),
                pltpu.make_async_copy(cx_hbm.at[idxi_v[b]], cvs[b][0],
                                      sems[b]),
                pltpu.make_async_copy(cy_hbm.at[idxi_v[b]], cvs[b][1],
                                      sems[b]),
                pltpu.make_async_copy(cz_hbm.at[idxi_v[b]], cvs[b][2],
                                      sems[b]),
                pltpu.make_async_copy(cx_hbm.at[idxj_v[b]], cvs[b][3],
                                      sems[b]),
                pltpu.make_async_copy(cy_hbm.at[idxj_v[b]], cvs[b][4],
                                      sems[b]),
                pltpu.make_async_copy(cz_hbm.at[idxj_v[b]], cvs[b][5],
                                      sems[b]),
            ]

        def start(t, b):
            base = t * CHUNK
            pltpu.sync_copy(idxj_hbm.at[pl.ds(base, CHUNK)], idxj_v[b])
            pltpu.sync_copy(idxi_hbm.at[pl.ds(base, CHUNK)], idxi_v[b])
            for cp in copies(b):
                cp.start()

        def finish(t, b):
            base = t * CHUNK
            for cp in copies(b):
                cp.wait()
            for g in range(CHUNK // 16):
                s = pl.ds(g * 16, 16)
                tb_v[b][0, s] = cvs[b][3][s] - cvs[b][0][s]
                tb_v[b][1, s] = cvs[b][4][s] - cvs[b][1][s]
                tb_v[b][2, s] = cvs[b][5][s] - cvs[b][2][s]
            pltpu.sync_copy(rows_v[b], gj_hbm.at[pl.ds(base, CHUNK)])
            pltpu.sync_copy(tb_v[b], rijt_hbm.at[:, pl.ds(base, CHUNK)])

        pairs = base_steps // 2
        rem = base_steps % 2
        tile = lambda i: wid + i * nw
        start(tile(0), 0)

        def pair_body(p, carry):
            i = 2 * p
            start(tile(i + 1), 1)
            finish(tile(i), 0)
            if rem == 1:
                start(tile(i + 2), 0)
            else:
                @pl.when(i + 2 < base_steps)
                def _s():
                    start(tile(i + 2), 0)
            finish(tile(i + 1), 1)
            return carry

        lax.fori_loop(0, pairs, pair_body, 0)
        if rem == 1:
            finish(tile(base_steps - 1), 0)

        @pl.when(wid < extra)
        def _tail():
            t = wid + base_steps * nw
            start(t, 0)
            finish(t, 0)

    return edge_gather


def _dot00(a, b):
    return lax.dot_general(a, b, (((0,), (0,)), ((), ())),
                           preferred_element_type=jnp.float32)


def _edge_body(gj_ref, rijt_ref, wr0_ref, wr1_ref, br0_ref, br1_ref,
               out0_ref, out1_ref):
    c_out = wr0_ref.shape[1]
    gt = lax.bitcast_convert_type(jnp.transpose(gj_ref[...]),
                                  jnp.uint32)
    hi = lax.bitcast_convert_type(gt & jnp.uint32(0xFFFF0000), jnp.float32)
    lo = lax.bitcast_convert_type(gt << 16, jnp.float32)
    g0t = hi[0:c_out, :]
    g1 = (hi[c_out:2 * c_out, :],
          lo[0:c_out, :],
          lo[c_out:2 * c_out, :])
    rijt = rijt_ref[...]
    be = rijt.shape[1]
    d2 = jnp.sum(rijt * rijt, axis=0, keepdims=True) + 1e-6
    d = jnp.sqrt(d2)
    rinv = 1.0 / d
    centers = lax.broadcasted_iota(jnp.int32, (16, be), 0).astype(
        jnp.float32) * (CUTOFF / 15.0)
    delta = d - centers
    rbf = jnp.exp(-GAMMA * delta * delta)
    fn0 = _dot00(wr0_ref[...], rbf) + br0_ref[...]
    fn1 = _dot00(wr1_ref[...], rbf) + br1_ref[...]
    acc = None
    for a in range(3):
        u_a = rijt[a:a + 1, :] * rinv
        out1_ref[pl.ds(a * c_out, c_out), :] = (g0t * u_a * fn1
                                                + g1[a] * fn0)
        ga_ua = g1[a] * u_a
        acc = ga_ua if acc is None else acc + ga_ua
    out0_ref[...] = g0t * fn0 + acc * fn1


def _edge_stage(gj, rijt, wr0, wr1, br0c, br1c, be):
    e = gj.shape[0]
    width = gj.shape[1]
    c_out = wr0.shape[1]
    grid = e // be
    full = lambda a: pl.BlockSpec(a.shape, lambda i: (0, 0))
    return pl.pallas_call(
        _edge_body,
        grid=(grid,),
        in_specs=[
            pl.BlockSpec((be, width), lambda i: (i, 0)),
            pl.BlockSpec((8, be), lambda i: (0, i)),
            full(wr0), full(wr1), full(br0c), full(br1c),
        ],
        out_specs=[
            pl.BlockSpec((c_out, be), lambda i: (0, i)),
            pl.BlockSpec((3 * c_out, be), lambda i: (0, i)),
        ],
        out_shape=[
            jax.ShapeDtypeStruct((c_out, e), jnp.float32),
            jax.ShapeDtypeStruct((3 * c_out, e), jnp.float32),
        ],
    )(gj, rijt, wr0, wr1, br0c, br1c)


def kernel(node_0, node_1, coord, idx_i, idx_j, W0, b0, W1, Wr0, br0, Wr1,
           br1):
    n, c_in = node_0.shape
    c_out = W0.shape[1]
    e = idx_i.shape[0]

    n1p = jnp.transpose(node_1, (2, 0, 1))
    coordt = jnp.transpose(coord)
    br0c = br0[:, None]
    br1c = br1[:, None]
    idxj32 = idx_j.astype(jnp.int32)
    idxi32 = idx_i.astype(jnp.int32)

    table = _build_table(node_0, n1p, W0, b0[None, :], W1, bn=1000)

    gj, rijt = _make_edge_gather(e, 2 * c_out)(
        table, coordt[0], coordt[1], coordt[2], idxi32, idxj32)

    out0t, out1t = _edge_stage(gj, rijt, Wr0, Wr1, br0c, br1c, be=6400)
    out0 = out0t.T
    out1 = out1t.reshape(3, c_out, e).transpose(2, 1, 0)
    return out0, out1
